# baseline (speedup 1.0000x reference)
import functools

import jax
import jax.numpy as jnp
from jax import lax
from jax.experimental import pallas as pl
from jax.experimental.pallas import tpu as pltpu
from jax.experimental.pallas import tpu_sc as plsc

_B = 4096
_D = 776
_DPAD = 896
_K = 18000
_KBLK = 368
_SUBS = 7
_CHUNK = _KBLK * _SUBS
_NBLK = 49
_IMAX = 2**31 - 1
_NC = 2
_B2 = _B // _NC


def _bf16_round(x):
    return x.astype(jnp.bfloat16).astype(jnp.float32)


def _argmin_body(zf_ref, zsq_ref, emb_ref, sqn_ref, idx_ref, loss_ref,
                 cm_ref, ci_ref, gm_ref, gde_ref):
    step = pl.program_id(1)
    sub = lax.rem(step, _SUBS)
    chunk = lax.div(step, _SUBS)

    scores = lax.dot_general(zf_ref[...], emb_ref[...],
                             (((1,), (1,)), ((), ())),
                             preferred_element_type=jnp.float32)
    col = lax.broadcasted_iota(jnp.int32, (_B2, _KBLK), 1) + step * _KBLK
    srow = sqn_ref[pl.ds(step, 1), :]
    d = (zsq_ref[...] + srow) - 2.0 * scores
    d = jnp.where(col < _K, d, jnp.float32(jnp.inf))
    m = jnp.min(d, axis=1, keepdims=True)
    li = jnp.min(jnp.where(d == m, col, jnp.int32(_IMAX)), axis=1, keepdims=True)

    @pl.when(sub == 0)
    def _():
        cm_ref[...] = m
        ci_ref[...] = li

    @pl.when(sub > 0)
    def _():
        upd = m < cm_ref[...]
        ci_ref[...] = jnp.where(upd, li, ci_ref[...])
        cm_ref[...] = jnp.where(upd, m, cm_ref[...])

    @pl.when(sub == _SUBS - 1)
    def _():
        @pl.when(chunk == 0)
        def _():
            gm_ref[...] = _bf16_round(cm_ref[...])
            gde_ref[...] = cm_ref[...]
            idx_ref[...] = ci_ref[...]

        @pl.when(chunk > 0)
        def _():
            upd = cm_ref[...] < gm_ref[...]
            idx_ref[...] = jnp.where(upd, ci_ref[...], idx_ref[...])
            gde_ref[...] = jnp.where(upd, cm_ref[...], gde_ref[...])
            gm_ref[...] = _bf16_round(jnp.where(upd, cm_ref[...], gm_ref[...]))

    @pl.when(step == _NBLK - 1)
    def _():
        part = 2.0 * jnp.sum(gde_ref[...], keepdims=True) / (_B * _D)
        loss_ref[...] = jnp.broadcast_to(part, (8, 128))


def _stage1(zf, zsq, emb, sqn):
    return pl.pallas_call(
        _argmin_body,
        grid=(_NC, _NBLK),
        in_specs=[
            pl.BlockSpec((_B2, _D), lambda i, k: (i, 0)),
            pl.BlockSpec((_B2, 1), lambda i, k: (i, 0)),
            pl.BlockSpec((_KBLK, _D), lambda i, k: (k, 0)),
            pl.BlockSpec((_NBLK, _KBLK), lambda i, k: (0, 0)),
        ],
        out_specs=[
            pl.BlockSpec((_B2, 1), lambda i, k: (i, 0)),
            pl.BlockSpec((8, 128), lambda i, k: (i, 0)),
        ],
        out_shape=[
            jax.ShapeDtypeStruct((_B, 1), jnp.int32),
            jax.ShapeDtypeStruct((_NC * 8, 128), jnp.float32),
        ],
        scratch_shapes=[
            pltpu.VMEM((_B2, 1), jnp.float32),
            pltpu.VMEM((_B2, 1), jnp.int32),
            pltpu.VMEM((_B2, 1), jnp.float32),
            pltpu.VMEM((_B2, 1), jnp.float32),
        ],
        compiler_params=pltpu.CompilerParams(
            dimension_semantics=("parallel", "arbitrary")),
    )(zf, zsq, emb, sqn)


def _sc_gather(table, idx):
    mesh = plsc.VectorSubcoreMesh(core_axis_name="c", subcore_axis_name="s")
    nw = mesh.num_cores * mesh.num_subcores
    bpw = _B // nw

    @functools.partial(
        pl.kernel,
        out_type=jax.ShapeDtypeStruct((_B, _DPAD), jnp.float32),
        mesh=mesh,
        scratch_types=[
            pltpu.VMEM((bpw,), jnp.int32),
            pltpu.VMEM((bpw, _DPAD), jnp.float32),
            pltpu.SemaphoreType.DMA,
        ],
    )
    def gather_kernel(table_hbm, idx_hbm, out_hbm, idx_v, rows_v, sem):
        wid = lax.axis_index("s") * mesh.num_cores + lax.axis_index("c")
        base = wid * bpw
        pltpu.sync_copy(idx_hbm.at[pl.ds(base, bpw)], idx_v)
        pltpu.async_copy(table_hbm.at[idx_v], rows_v, sem).wait()
        pltpu.sync_copy(rows_v, out_hbm.at[pl.ds(base, bpw)])

    return gather_kernel(table, idx)


def kernel(text_features, graph_features, codebook):
    z = jnp.concatenate([text_features, graph_features], axis=-1)
    zf = z / jnp.clip(jnp.linalg.norm(z, axis=-1, keepdims=True), 1e-12)
    emb = codebook / jnp.clip(jnp.linalg.norm(codebook, axis=-1, keepdims=True), 1e-12)
    zsq = jnp.sum(zf ** 2, axis=1, keepdims=True)
    sqn = jnp.pad(jnp.sum(emb ** 2, axis=1),
                  (0, _NBLK * _KBLK - _K)).reshape(_NBLK, _KBLK)
    idx2, loss = _stage1(zf, zsq, emb, sqn)
    idx = idx2.reshape(_B)
    embn = jnp.pad(emb, ((0, 0), (0, _DPAD - _D)))
    quant = _sc_gather(embn, idx)[:, :_D]
    return quant, loss[0, 0] + loss[8, 0], idx


# speedup vs baseline: 1.0616x; 1.0616x over previous
import functools

import jax
import jax.numpy as jnp
from jax import lax
from jax.experimental import pallas as pl
from jax.experimental.pallas import tpu as pltpu
from jax.experimental.pallas import tpu_sc as plsc

_B = 4096
_D = 776
_DPAD = 896
_K = 18000
_KBLK = 368
_SUBS = 7
_CHUNK = _KBLK * _SUBS
_NBLK = 49
_IMAX = 2**31 - 1


def _bf16_round(x):
    return x.astype(jnp.bfloat16).astype(jnp.float32)


def _argmin_body(zf_ref, zsq_ref, emb_ref, sqn_ref, embn_ref, idx_ref,
                 loss_ref, cm_ref, ci_ref, gm_ref, gde_ref):
    step = pl.program_id(0)
    sub = lax.rem(step, _SUBS)
    chunk = lax.div(step, _SUBS)

    emb = emb_ref[...]
    embn_ref[...] = jnp.concatenate(
        [emb, jnp.zeros((_KBLK, _DPAD - _D), jnp.float32)], axis=1)

    scores = lax.dot_general(zf_ref[...], emb, (((1,), (1,)), ((), ())),
                             preferred_element_type=jnp.float32)
    col = lax.broadcasted_iota(jnp.int32, (_B, _KBLK), 1) + step * _KBLK
    srow = sqn_ref[pl.ds(step, 1), :]
    d = (zsq_ref[...] + srow) - 2.0 * scores
    d = jnp.where(col < _K, d, jnp.float32(jnp.inf))
    m = jnp.min(d, axis=1, keepdims=True)
    li = jnp.min(jnp.where(d == m, col, jnp.int32(_IMAX)), axis=1, keepdims=True)

    @pl.when(sub == 0)
    def _():
        cm_ref[...] = m
        ci_ref[...] = li

    @pl.when(sub > 0)
    def _():
        upd = m < cm_ref[...]
        ci_ref[...] = jnp.where(upd, li, ci_ref[...])
        cm_ref[...] = jnp.where(upd, m, cm_ref[...])

    @pl.when(sub == _SUBS - 1)
    def _():
        @pl.when(chunk == 0)
        def _():
            gm_ref[...] = _bf16_round(cm_ref[...])
            gde_ref[...] = cm_ref[...]
            idx_ref[...] = ci_ref[...]

        @pl.when(chunk > 0)
        def _():
            upd = cm_ref[...] < gm_ref[...]
            idx_ref[...] = jnp.where(upd, ci_ref[...], idx_ref[...])
            gde_ref[...] = jnp.where(upd, cm_ref[...], gde_ref[...])
            gm_ref[...] = _bf16_round(jnp.where(upd, cm_ref[...], gm_ref[...]))

    @pl.when(step == _NBLK - 1)
    def _():
        loss_ref[...] = 2.0 * jnp.sum(gde_ref[...], keepdims=True) / (_B * _D)


def _stage1(zf, zsq, emb, sqn):
    return pl.pallas_call(
        _argmin_body,
        grid=(_NBLK,),
        in_specs=[
            pl.BlockSpec((_B, _D), lambda k: (0, 0)),
            pl.BlockSpec((_B, 1), lambda k: (0, 0)),
            pl.BlockSpec((_KBLK, _D), lambda k: (k, 0)),
            pl.BlockSpec((_NBLK, _KBLK), lambda k: (0, 0)),
        ],
        out_specs=[
            pl.BlockSpec((_KBLK, _DPAD), lambda k: (k, 0)),
            pl.BlockSpec((_B, 1), lambda k: (0, 0)),
            pl.BlockSpec((1, 1), lambda k: (0, 0)),
        ],
        out_shape=[
            jax.ShapeDtypeStruct((_K, _DPAD), jnp.float32),
            jax.ShapeDtypeStruct((_B, 1), jnp.int32),
            jax.ShapeDtypeStruct((1, 1), jnp.float32),
        ],
        scratch_shapes=[
            pltpu.VMEM((_B, 1), jnp.float32),
            pltpu.VMEM((_B, 1), jnp.int32),
            pltpu.VMEM((_B, 1), jnp.float32),
            pltpu.VMEM((_B, 1), jnp.float32),
        ],
        compiler_params=pltpu.CompilerParams(
            dimension_semantics=("arbitrary",)),
    )(zf, zsq, emb, sqn)


def _sc_gather(table, idx):
    mesh = plsc.VectorSubcoreMesh(core_axis_name="c", subcore_axis_name="s")
    nw = mesh.num_cores * mesh.num_subcores
    bpw = _B // nw

    @functools.partial(
        pl.kernel,
        out_type=jax.ShapeDtypeStruct((_B, _DPAD), jnp.float32),
        mesh=mesh,
        scratch_types=[
            pltpu.VMEM((bpw,), jnp.int32),
            pltpu.VMEM((bpw, _DPAD), jnp.float32),
            pltpu.SemaphoreType.DMA,
        ],
    )
    def gather_kernel(table_hbm, idx_hbm, out_hbm, idx_v, rows_v, sem):
        wid = lax.axis_index("s") * mesh.num_cores + lax.axis_index("c")
        base = wid * bpw
        pltpu.sync_copy(idx_hbm.at[pl.ds(base, bpw)], idx_v)
        pltpu.async_copy(table_hbm.at[idx_v], rows_v, sem).wait()
        pltpu.sync_copy(rows_v, out_hbm.at[pl.ds(base, bpw)])

    return gather_kernel(table, idx)


def kernel(text_features, graph_features, codebook):
    z = jnp.concatenate([text_features, graph_features], axis=-1)
    zf = z / jnp.clip(jnp.linalg.norm(z, axis=-1, keepdims=True), 1e-12)
    emb = codebook / jnp.clip(jnp.linalg.norm(codebook, axis=-1, keepdims=True), 1e-12)
    zsq = jnp.sum(zf ** 2, axis=1, keepdims=True)
    sqn = jnp.pad(jnp.sum(emb ** 2, axis=1),
                  (0, _NBLK * _KBLK - _K)).reshape(_NBLK, _KBLK)
    embn, idx2, loss = _stage1(zf, zsq, emb, sqn)
    idx = idx2.reshape(_B)
    quant = _sc_gather(embn, idx)[:, :_D]
    return quant, loss[0, 0], idx
